# baseline (speedup 1.0000x reference)
import dataclasses
import functools

import jax
import jax.numpy as jnp
from jax import lax
from jax.experimental import pallas as pl
from jax.experimental.pallas import tpu as pltpu
from jax.experimental.pallas import tpu_sc as plsc

N = 10000
E = 320000
D = 128
R = 8
RB = 4
NC = 2
NS = 16
L = 16
NW = NC * NS

CH = 5120
CHB = 1024
RNP = NS * CH
K = 80
EPW = E // NW
NB_AGG = EPW // K
EPT_H = E // NS
NB_H = EPT_H // K
ROWS_PT = N // NS
PKB = 16384

BN = 1000
NBK = N // BN

f32 = jnp.float32
i32 = jnp.int32
bf16 = jnp.bfloat16

_mesh = plsc.VectorSubcoreMesh(core_axis_name="c", subcore_axis_name="s")

_sc_params = pltpu.CompilerParams()
if "needs_layout_passes" in pltpu.CompilerParams.__dataclass_fields__:
    _sc_params = dataclasses.replace(_sc_params, needs_layout_passes=False)
if "use_tc_tiling_on_sc" in pltpu.CompilerParams.__dataclass_fields__:
    _sc_params = dataclasses.replace(_sc_params, use_tc_tiling_on_sc=False)


@functools.partial(
    pl.kernel,
    out_type=jax.ShapeDtypeStruct((RNP, L), f32),
    mesh=_mesh,
    compiler_params=_sc_params,
    scratch_types=[
        pltpu.VMEM_SHARED((RNP,), f32),
        pltpu.VMEM((NB_H, K), i32),
        pltpu.VMEM((K,), f32),
        pltpu.VMEM((CH,), f32),
        pltpu.VMEM((CHB, L), f32),
        pltpu.SemaphoreType.DMA,
    ],
)
def _hist_weights(gdst3_hbm, inv16_hbm, cnt_sp, gidx2_v, ones_v, inv_v,
                  inv16_v, sem):
    cid = lax.axis_index("c")
    sid = lax.axis_index("s")

    @pl.when(cid == 0)
    def _():
        pltpu.sync_copy(gdst3_hbm.at[sid], gidx2_v)

        @pl.loop(0, CH // L)
        def _(i):
            inv_v[pl.ds(i * L, L)] = jnp.zeros((L,), f32)

        pltpu.sync_copy(inv_v, cnt_sp.at[pl.ds(sid * CH, CH)])

        @pl.loop(0, K // L)
        def _(i):
            ones_v[pl.ds(i * L, L)] = jnp.ones((L,), f32)

        plsc.subcore_barrier()

        @pl.loop(0, NB_H // 10)
        def _(i):
            for j in range(10):
                pltpu.make_async_copy(
                    ones_v, cnt_sp.at[gidx2_v.at[i * 10 + j]], sem,
                ).start(add=True)
            for j in range(10):
                pltpu.make_async_copy(
                    ones_v, cnt_sp.at[gidx2_v.at[i * 10 + j]], sem,
                ).wait()

        plsc.subcore_barrier()

        pltpu.sync_copy(cnt_sp.at[pl.ds(sid * CH, CH)], inv_v)

        @pl.loop(0, CH // L)
        def _(i):
            v = inv_v[pl.ds(i * L, L)]
            inv_v[pl.ds(i * L, L)] = 1.0 / jnp.maximum(v, 1.0)

        for c in range(CH // CHB):
            @pl.loop(0, CHB)
            def _(j):
                inv16_v[j, :] = plsc.load_gather(
                    inv_v, [jnp.full((L,), c * CHB + j, i32)])

            pltpu.sync_copy(inv16_v,
                            inv16_hbm.at[pl.ds(sid * CH + c * CHB, CHB)])


@functools.partial(
    pl.kernel,
    out_type=[jax.ShapeDtypeStruct((N, D), f32),
              jax.ShapeDtypeStruct((N, D), f32)],
    mesh=_mesh,
    compiler_params=_sc_params,
    scratch_types=[
        pltpu.VMEM_SHARED((N, D), f32),
        pltpu.VMEM((NB_AGG, K), i32),
        pltpu.VMEM((K,), i32),
        pltpu.VMEM((K,), i32),
        pltpu.VMEM((K,), i32),
        pltpu.VMEM((K,), i32),
        pltpu.VMEM((K,), i32),
        pltpu.VMEM((K,), i32),
        pltpu.VMEM((K, D), bf16),
        pltpu.VMEM((K, D), bf16),
        pltpu.VMEM((K, D), f32),
        pltpu.VMEM((K, D), f32),
        pltpu.VMEM((K, L), f32),
        pltpu.VMEM((K, L), f32),
        pltpu.SemaphoreType.DMA,
        pltpu.SemaphoreType.DMA,
        pltpu.SemaphoreType.DMA,
        pltpu.SemaphoreType.DMA,
    ],
)
def _agg(h_hbm, epk3_hbm, inv16_hbm, p0_hbm, p1_hbm,
         acc_sp, epk2_v, gidx_a, gidx_b, widx_a, widx_b, didx_a, didx_b,
         rbf_a, rbf_b, rf_a, rf_b, w16_a, w16_b, sg_a, sg_b, ss_a, ss_b):
    cid = lax.axis_index("c")
    sid = lax.axis_index("s")
    wid = cid * NS + sid

    pltpu.sync_copy(epk3_hbm.at[wid], epk2_v)

    @pl.loop(0, K)
    def _(i):
        for j in range(D // L):
            rf_a[i, pl.ds(j * L, L)] = jnp.zeros((L,), f32)

    for j in range(7):
        pltpu.sync_copy(rf_a, acc_sp.at[pl.ds(sid * ROWS_PT + j * K, K)])
    pltpu.sync_copy(rf_a, acc_sp.at[pl.ds(sid * ROWS_PT + ROWS_PT - K, K)])

    plsc.subcore_barrier()

    def mk_idx(b, gidx, widx, didx):
        @pl.loop(0, K // L)
        def _(j):
            p = epk2_v[b, pl.ds(j * L, L)]
            d = jnp.bitwise_and(p, PKB - 1)
            g = jnp.right_shift(p, 14)
            t = ((g.astype(f32) + 0.5) * (1.0 / N)).astype(i32)
            gidx[pl.ds(j * L, L)] = g
            didx[pl.ds(j * L, L)] = d
            widx[pl.ds(j * L, L)] = t * N + d

    def g_start(gidx, widx, rbf, w16, sem):
        pltpu.make_async_copy(h_hbm.at[gidx], rbf, sem).start()
        pltpu.make_async_copy(inv16_hbm.at[widx], w16, sem).start()

    def g_wait(gidx, widx, rbf, w16, sem):
        pltpu.make_async_copy(h_hbm.at[gidx], rbf, sem).wait()
        pltpu.make_async_copy(inv16_hbm.at[widx], w16, sem).wait()

    def s_start(rf, didx, sem):
        pltpu.make_async_copy(rf, acc_sp.at[didx], sem).start(add=True)

    def s_wait(rf, didx, sem):
        pltpu.make_async_copy(rf, acc_sp.at[didx], sem).wait()

    iot = lax.iota(i32, L)
    idx_ev = [32 * c + 2 * iot for c in range(D // 32)]
    idx_od = [32 * c + 1 + 2 * iot for c in range(D // 32)]

    def scale(rbf, rf, w16):
        @pl.loop(0, K)
        def _(k):
            wv = w16[k, :]
            kv = jnp.full((L,), k, i32)
            for c in range(D // 32):
                xb = rbf[k, pl.ds(32 * c, 32)]
                ev, od = plsc.unpack(xb, format=plsc.PackFormat.INTERLEAVED)
                plsc.store_scatter(rf, [kv, idx_ev[c]], ev * wv)
                plsc.store_scatter(rf, [kv, idx_od[c]], od * wv)

    mk_idx(0, gidx_a, widx_a, didx_a)
    g_start(gidx_a, widx_a, rbf_a, w16_a, sg_a)
    mk_idx(1, gidx_b, widx_b, didx_b)
    g_start(gidx_b, widx_b, rbf_b, w16_b, sg_b)

    @pl.loop(0, NB_AGG // 2)
    def _(i):
        b0 = 2 * i
        g_wait(gidx_a, widx_a, rbf_a, w16_a, sg_a)
        scale(rbf_a, rf_a, w16_a)
        s_start(rf_a, didx_a, ss_a)

        g_wait(gidx_b, widx_b, rbf_b, w16_b, sg_b)
        scale(rbf_b, rf_b, w16_b)
        s_start(rf_b, didx_b, ss_b)

        s_wait(rf_a, didx_a, ss_a)
        mk_idx(b0 + 2, gidx_a, widx_a, didx_a)
        g_start(gidx_a, widx_a, rbf_a, w16_a, sg_a)

        s_wait(rf_b, didx_b, ss_b)

        @pl.when(b0 + 3 < NB_AGG)
        def _():
            mk_idx(b0 + 3, gidx_b, widx_b, didx_b)
            g_start(gidx_b, widx_b, rbf_b, w16_b, sg_b)

    g_wait(gidx_a, widx_a, rbf_a, w16_a, sg_a)
    scale(rbf_a, rf_a, w16_a)
    pltpu.sync_copy(rf_a, acc_sp.at[didx_a], add=True)

    plsc.subcore_barrier()

    @pl.when(cid == 0)
    def _():
        pltpu.sync_copy(acc_sp.at[pl.ds(sid * ROWS_PT, ROWS_PT)],
                        p0_hbm.at[pl.ds(sid * ROWS_PT, ROWS_PT)])

    @pl.when(cid == 1)
    def _():
        pltpu.sync_copy(acc_sp.at[pl.ds(sid * ROWS_PT, ROWS_PT)],
                        p1_hbm.at[pl.ds(sid * ROWS_PT, ROWS_PT)])


def _mk_w(comp_blk, basis):
    c = comp_blk[0]
    w = c[0:1, 0:1] * basis[0]
    for b in range(1, RB):
        w = w + c[0:1, b:b + 1] * basis[b]
    return w


def _h8_spec():
    return pl.BlockSpec((1, BN, D),
                        lambda i, r: (jnp.minimum(r, R - 1), i, 0))


def _prep1_body(x_ref, comp_ref, basis_ref, root_ref, bias_ref,
                h8_ref, rt_ref):
    r = pl.program_id(1)

    @pl.when(r < R)
    def _():
        w = _mk_w(comp_ref[...], basis_ref[...])
        h8_ref[0] = jnp.dot(x_ref[...], w,
                            preferred_element_type=f32).astype(bf16)

    @pl.when(r == R)
    def _():
        rt_ref[...] = (jnp.dot(x_ref[...], root_ref[...],
                               preferred_element_type=f32) + bias_ref[...])


_prep1 = pl.pallas_call(
    _prep1_body,
    grid=(NBK, R + 1),
    in_specs=[
        pl.BlockSpec((BN, D), lambda i, r: (i, 0)),
        pl.BlockSpec((1, 1, RB), lambda i, r: (r, 0, 0)),
        pl.BlockSpec((RB, D, D), lambda i, r: (0, 0, 0)),
        pl.BlockSpec((D, D), lambda i, r: (0, 0)),
        pl.BlockSpec((1, D), lambda i, r: (0, 0)),
    ],
    out_specs=[_h8_spec(), pl.BlockSpec((BN, D), lambda i, r: (i, 0))],
    out_shape=[jax.ShapeDtypeStruct((R, N, D), bf16),
               jax.ShapeDtypeStruct((N, D), f32)],
)


def _prep2_body(p0_ref, p1_ref, rt_ref, comp_ref, basis_ref, root_ref,
                bias_ref, h8_ref, rt2_ref, h_v):
    r = pl.program_id(1)

    @pl.when(r == 0)
    def _():
        h_v[...] = jax.nn.sigmoid(p0_ref[...] + p1_ref[...] + rt_ref[...])

    @pl.when(r < R)
    def _():
        w = _mk_w(comp_ref[...], basis_ref[...])
        h8_ref[0] = jnp.dot(h_v[...], w,
                            preferred_element_type=f32).astype(bf16)

    @pl.when(r == R)
    def _():
        rt2_ref[...] = (jnp.dot(h_v[...], root_ref[...],
                                preferred_element_type=f32) + bias_ref[...])


_prep2 = pl.pallas_call(
    _prep2_body,
    grid=(NBK, R + 1),
    in_specs=[
        pl.BlockSpec((BN, D), lambda i, r: (i, 0)),
        pl.BlockSpec((BN, D), lambda i, r: (i, 0)),
        pl.BlockSpec((BN, D), lambda i, r: (i, 0)),
        pl.BlockSpec((1, 1, RB), lambda i, r: (r, 0, 0)),
        pl.BlockSpec((RB, D, D), lambda i, r: (0, 0, 0)),
        pl.BlockSpec((D, D), lambda i, r: (0, 0)),
        pl.BlockSpec((1, D), lambda i, r: (0, 0)),
    ],
    out_specs=[_h8_spec(), pl.BlockSpec((BN, D), lambda i, r: (i, 0))],
    out_shape=[jax.ShapeDtypeStruct((R, N, D), bf16),
               jax.ShapeDtypeStruct((N, D), f32)],
    scratch_shapes=[pltpu.VMEM((BN, D), f32)],
)


def _combine_body(p0_ref, p1_ref, rt_ref, out_ref):
    out_ref[...] = jax.nn.sigmoid(p0_ref[...] + p1_ref[...] + rt_ref[...])


_combine = pl.pallas_call(
    _combine_body,
    grid=(NBK,),
    in_specs=[
        pl.BlockSpec((BN, D), lambda i: (i, 0)),
        pl.BlockSpec((BN, D), lambda i: (i, 0)),
        pl.BlockSpec((BN, D), lambda i: (i, 0)),
    ],
    out_specs=pl.BlockSpec((BN, D), lambda i: (i, 0)),
    out_shape=jax.ShapeDtypeStruct((N, D), f32),
)


def kernel(x, edge_index, edge_type, basis1, comp1, root1, bias1,
           basis2, comp2, root2, bias2):
    src = edge_index[0]
    dst = edge_index[1]
    gsrc = edge_type * N + src
    gdst = edge_type * N + dst
    epk3 = (gsrc * PKB + dst).reshape(NW, NB_AGG, K)
    gdst3h = gdst.reshape(NS, NB_H, K)

    inv16 = _hist_weights(gdst3h)

    pad = jnp.zeros((1, 1, RB), f32)
    comp1p = jnp.concatenate([comp1.reshape(R, 1, RB), pad], axis=0)
    comp2p = jnp.concatenate([comp2.reshape(R, 1, RB), pad], axis=0)

    h8_1, rt1 = _prep1(x, comp1p, basis1, root1, bias1.reshape(1, D))
    p0_1, p1_1 = _agg(h8_1.reshape(R * N, D), epk3, inv16)

    h8_2, rt2 = _prep2(p0_1, p1_1, rt1, comp2p, basis2, root2,
                       bias2.reshape(1, D))
    p0_2, p1_2 = _agg(h8_2.reshape(R * N, D), epk3, inv16)

    return _combine(p0_2, p1_2, rt2)


# speedup vs baseline: 1.8909x; 1.8909x over previous
import dataclasses
import functools

import jax
import jax.numpy as jnp
from jax import lax
from jax.experimental import pallas as pl
from jax.experimental.pallas import tpu as pltpu
from jax.experimental.pallas import tpu_sc as plsc

N = 10000
E = 320000
D = 128
R = 8
RB = 4
NC = 2
NS = 16
L = 16
NW = NC * NS

CH = 5120
CHB = 1024
RNP = NS * CH
K = 80
EPW = E // NW
NB_AGG = EPW // K
EPT_H = E // NS
NB_H = EPT_H // K
ROWS_PT = N // NS
PKB = 16384

BN = 1000
NBK = N // BN

f32 = jnp.float32
i32 = jnp.int32
bf16 = jnp.bfloat16

_mesh = plsc.VectorSubcoreMesh(core_axis_name="c", subcore_axis_name="s")

_sc_params = pltpu.CompilerParams()
if "needs_layout_passes" in pltpu.CompilerParams.__dataclass_fields__:
    _sc_params = dataclasses.replace(_sc_params, needs_layout_passes=False)
if "use_tc_tiling_on_sc" in pltpu.CompilerParams.__dataclass_fields__:
    _sc_params = dataclasses.replace(_sc_params, use_tc_tiling_on_sc=False)


@functools.partial(
    pl.kernel,
    out_type=jax.ShapeDtypeStruct((RNP, L), f32),
    mesh=_mesh,
    compiler_params=_sc_params,
    scratch_types=[
        pltpu.VMEM_SHARED((RNP,), f32),
        pltpu.VMEM((NB_H, K), i32),
        pltpu.VMEM((K,), f32),
        pltpu.VMEM((CH,), f32),
        pltpu.VMEM((CHB, L), f32),
        pltpu.SemaphoreType.DMA,
    ],
)
def _hist_weights(gdst3_hbm, inv16_hbm, cnt_sp, gidx2_v, ones_v, inv_v,
                  inv16_v, sem):
    cid = lax.axis_index("c")
    sid = lax.axis_index("s")

    @pl.when(cid == 0)
    def _():
        pltpu.sync_copy(gdst3_hbm.at[sid], gidx2_v)

        @pl.loop(0, CH // L)
        def _(i):
            inv_v[pl.ds(i * L, L)] = jnp.zeros((L,), f32)

        pltpu.sync_copy(inv_v, cnt_sp.at[pl.ds(sid * CH, CH)])

        @pl.loop(0, K // L)
        def _(i):
            ones_v[pl.ds(i * L, L)] = jnp.ones((L,), f32)

        plsc.subcore_barrier()

        @pl.loop(0, NB_H // 10)
        def _(i):
            for j in range(10):
                pltpu.make_async_copy(
                    ones_v, cnt_sp.at[gidx2_v.at[i * 10 + j]], sem,
                ).start(add=True)
            for j in range(10):
                pltpu.make_async_copy(
                    ones_v, cnt_sp.at[gidx2_v.at[i * 10 + j]], sem,
                ).wait()

        plsc.subcore_barrier()

        pltpu.sync_copy(cnt_sp.at[pl.ds(sid * CH, CH)], inv_v)

        @pl.loop(0, CH // L)
        def _(i):
            v = inv_v[pl.ds(i * L, L)]
            inv_v[pl.ds(i * L, L)] = 1.0 / jnp.maximum(v, 1.0)

        for c in range(CH // CHB):
            @pl.loop(0, CHB)
            def _(j):
                inv16_v[j, :] = plsc.load_gather(
                    inv_v, [jnp.full((L,), c * CHB + j, i32)])

            pltpu.sync_copy(inv16_v,
                            inv16_hbm.at[pl.ds(sid * CH + c * CHB, CHB)])


@functools.partial(
    pl.kernel,
    out_type=[jax.ShapeDtypeStruct((N, D), f32),
              jax.ShapeDtypeStruct((N, D), f32)],
    mesh=_mesh,
    compiler_params=_sc_params,
    scratch_types=[
        pltpu.VMEM_SHARED((N, D), f32),
        pltpu.VMEM((NB_AGG, K), i32),
        pltpu.VMEM((K,), i32),
        pltpu.VMEM((K,), i32),
        pltpu.VMEM((K,), i32),
        pltpu.VMEM((K,), i32),
        pltpu.VMEM((K,), i32),
        pltpu.VMEM((K,), i32),
        pltpu.VMEM((K, D), f32),
        pltpu.VMEM((K, D), f32),
        pltpu.VMEM((K, L), f32),
        pltpu.VMEM((K, L), f32),
        pltpu.SemaphoreType.DMA,
        pltpu.SemaphoreType.DMA,
        pltpu.SemaphoreType.DMA,
        pltpu.SemaphoreType.DMA,
    ],
)
def _agg(h_hbm, epk3_hbm, inv16_hbm, p0_hbm, p1_hbm,
         acc_sp, epk2_v, gidx_a, gidx_b, widx_a, widx_b, didx_a, didx_b,
         rows_a, rows_b, w16_a, w16_b, sg_a, sg_b, ss_a, ss_b):
    cid = lax.axis_index("c")
    sid = lax.axis_index("s")
    wid = cid * NS + sid

    pltpu.sync_copy(epk3_hbm.at[wid], epk2_v)

    @pl.loop(0, K)
    def _(i):
        for j in range(D // L):
            rows_a[i, pl.ds(j * L, L)] = jnp.zeros((L,), f32)

    for j in range(7):
        pltpu.sync_copy(rows_a, acc_sp.at[pl.ds(sid * ROWS_PT + j * K, K)])
    pltpu.sync_copy(rows_a, acc_sp.at[pl.ds(sid * ROWS_PT + ROWS_PT - K, K)])

    plsc.subcore_barrier()

    def mk_idx(b, gidx, widx, didx):
        @pl.loop(0, K // L)
        def _(j):
            p = epk2_v[b, pl.ds(j * L, L)]
            d = jnp.bitwise_and(p, PKB - 1)
            g = jnp.right_shift(p, 14)
            t = ((g.astype(f32) + 0.5) * (1.0 / N)).astype(i32)
            gidx[pl.ds(j * L, L)] = g
            didx[pl.ds(j * L, L)] = d
            widx[pl.ds(j * L, L)] = t * N + d

    def g_start(gidx, widx, rows, w16, sem):
        pltpu.make_async_copy(h_hbm.at[gidx], rows, sem).start()
        pltpu.make_async_copy(inv16_hbm.at[widx], w16, sem).start()

    def g_wait(gidx, widx, rows, w16, sem):
        pltpu.make_async_copy(h_hbm.at[gidx], rows, sem).wait()
        pltpu.make_async_copy(inv16_hbm.at[widx], w16, sem).wait()

    def s_start(rows, didx, sem):
        pltpu.make_async_copy(rows, acc_sp.at[didx], sem).start(add=True)

    def s_wait(rows, didx, sem):
        pltpu.make_async_copy(rows, acc_sp.at[didx], sem).wait()

    def scale(rows, w16):
        @plsc.parallel_loop(0, K, unroll=4)
        def _(k):
            wv = w16[k, :]
            for j in range(D // L):
                sl = (k, pl.ds(j * L, L))
                rows[sl] = rows[sl] * wv

    mk_idx(0, gidx_a, widx_a, didx_a)
    g_start(gidx_a, widx_a, rows_a, w16_a, sg_a)
    mk_idx(1, gidx_b, widx_b, didx_b)
    g_start(gidx_b, widx_b, rows_b, w16_b, sg_b)

    @pl.loop(0, NB_AGG // 2)
    def _(i):
        b0 = 2 * i
        g_wait(gidx_a, widx_a, rows_a, w16_a, sg_a)
        scale(rows_a, w16_a)
        s_start(rows_a, didx_a, ss_a)

        g_wait(gidx_b, widx_b, rows_b, w16_b, sg_b)
        scale(rows_b, w16_b)
        s_start(rows_b, didx_b, ss_b)

        s_wait(rows_a, didx_a, ss_a)
        mk_idx(b0 + 2, gidx_a, widx_a, didx_a)
        g_start(gidx_a, widx_a, rows_a, w16_a, sg_a)

        s_wait(rows_b, didx_b, ss_b)

        @pl.when(b0 + 3 < NB_AGG)
        def _():
            mk_idx(b0 + 3, gidx_b, widx_b, didx_b)
            g_start(gidx_b, widx_b, rows_b, w16_b, sg_b)

    g_wait(gidx_a, widx_a, rows_a, w16_a, sg_a)
    scale(rows_a, w16_a)
    pltpu.sync_copy(rows_a, acc_sp.at[didx_a], add=True)

    plsc.subcore_barrier()

    @pl.when(cid == 0)
    def _():
        pltpu.sync_copy(acc_sp.at[pl.ds(sid * ROWS_PT, ROWS_PT)],
                        p0_hbm.at[pl.ds(sid * ROWS_PT, ROWS_PT)])

    @pl.when(cid == 1)
    def _():
        pltpu.sync_copy(acc_sp.at[pl.ds(sid * ROWS_PT, ROWS_PT)],
                        p1_hbm.at[pl.ds(sid * ROWS_PT, ROWS_PT)])


def _mk_w(comp_blk, basis):
    c = comp_blk[0]
    w = c[0:1, 0:1] * basis[0]
    for b in range(1, RB):
        w = w + c[0:1, b:b + 1] * basis[b]
    return w


def _prep1_body(x_ref, comp_ref, basis_ref, root_ref, bias_ref, out_ref,
                w_sc, xb_sc):
    i = pl.program_id(0)
    r = pl.program_id(1)

    @pl.when(r == 0)
    def _():
        xb_sc[...] = x_ref[...].astype(bf16)

    @pl.when((i == 0) & (r < R))
    def _():
        w_sc[r] = _mk_w(comp_ref[...], basis_ref[...]).astype(bf16)

    @pl.when(r < R)
    def _():
        out_ref[0] = jnp.dot(xb_sc[...], w_sc[r],
                             preferred_element_type=f32)

    @pl.when(r == R)
    def _():
        out_ref[0] = (jnp.dot(x_ref[...], root_ref[...],
                              preferred_element_type=f32) + bias_ref[...])


_prep1 = pl.pallas_call(
    _prep1_body,
    grid=(NBK, R + 1),
    in_specs=[
        pl.BlockSpec((BN, D), lambda i, r: (i, 0)),
        pl.BlockSpec((1, 1, RB), lambda i, r: (r, 0, 0)),
        pl.BlockSpec((RB, D, D), lambda i, r: (0, 0, 0)),
        pl.BlockSpec((D, D), lambda i, r: (0, 0)),
        pl.BlockSpec((1, D), lambda i, r: (0, 0)),
    ],
    out_specs=pl.BlockSpec((1, BN, D), lambda i, r: (r, i, 0)),
    out_shape=jax.ShapeDtypeStruct((R + 1, N, D), f32),
    scratch_shapes=[pltpu.VMEM((R, D, D), bf16),
                    pltpu.VMEM((BN, D), bf16)],
)


def _prep2_body(p0_ref, p1_ref, rt_ref, comp_ref, basis_ref, root_ref,
                bias_ref, out_ref, w_sc, h_v, hb_sc):
    i = pl.program_id(0)
    r = pl.program_id(1)

    @pl.when(r == 0)
    def _():
        h = jax.nn.sigmoid(p0_ref[...] + p1_ref[...] + rt_ref[...])
        h_v[...] = h
        hb_sc[...] = h.astype(bf16)

    @pl.when((i == 0) & (r < R))
    def _():
        w_sc[r] = _mk_w(comp_ref[...], basis_ref[...]).astype(bf16)

    @pl.when(r < R)
    def _():
        out_ref[0] = jnp.dot(hb_sc[...], w_sc[r],
                             preferred_element_type=f32)

    @pl.when(r == R)
    def _():
        out_ref[0] = (jnp.dot(h_v[...], root_ref[...],
                              preferred_element_type=f32) + bias_ref[...])


_prep2 = pl.pallas_call(
    _prep2_body,
    grid=(NBK, R + 1),
    in_specs=[
        pl.BlockSpec((BN, D), lambda i, r: (i, 0)),
        pl.BlockSpec((BN, D), lambda i, r: (i, 0)),
        pl.BlockSpec((BN, D), lambda i, r: (i, 0)),
        pl.BlockSpec((1, 1, RB), lambda i, r: (r, 0, 0)),
        pl.BlockSpec((RB, D, D), lambda i, r: (0, 0, 0)),
        pl.BlockSpec((D, D), lambda i, r: (0, 0)),
        pl.BlockSpec((1, D), lambda i, r: (0, 0)),
    ],
    out_specs=pl.BlockSpec((1, BN, D), lambda i, r: (r, i, 0)),
    out_shape=jax.ShapeDtypeStruct((R + 1, N, D), f32),
    scratch_shapes=[pltpu.VMEM((R, D, D), bf16),
                    pltpu.VMEM((BN, D), f32),
                    pltpu.VMEM((BN, D), bf16)],
)


def _combine_body(p0_ref, p1_ref, rt_ref, out_ref):
    out_ref[...] = jax.nn.sigmoid(p0_ref[...] + p1_ref[...] + rt_ref[...])


_combine = pl.pallas_call(
    _combine_body,
    grid=(NBK,),
    in_specs=[
        pl.BlockSpec((BN, D), lambda i: (i, 0)),
        pl.BlockSpec((BN, D), lambda i: (i, 0)),
        pl.BlockSpec((BN, D), lambda i: (i, 0)),
    ],
    out_specs=pl.BlockSpec((BN, D), lambda i: (i, 0)),
    out_shape=jax.ShapeDtypeStruct((N, D), f32),
)


def kernel(x, edge_index, edge_type, basis1, comp1, root1, bias1,
           basis2, comp2, root2, bias2):
    src = edge_index[0]
    dst = edge_index[1]
    gsrc = edge_type * N + src
    gdst = edge_type * N + dst
    epk3 = (gsrc * PKB + dst).reshape(NW, NB_AGG, K)
    gdst3h = gdst.reshape(NS, NB_H, K)

    inv16 = _hist_weights(gdst3h)

    pad = jnp.zeros((1, 1, RB), f32)
    comp1p = jnp.concatenate([comp1.reshape(R, 1, RB), pad], axis=0)
    comp2p = jnp.concatenate([comp2.reshape(R, 1, RB), pad], axis=0)

    h9_1 = _prep1(x, comp1p, basis1, root1, bias1.reshape(1, D))
    p0_1, p1_1 = _agg(h9_1.reshape((R + 1) * N, D), epk3, inv16)

    h9_2 = _prep2(p0_1, p1_1, h9_1[R], comp2p, basis2, root2,
                  bias2.reshape(1, D))
    p0_2, p1_2 = _agg(h9_2.reshape((R + 1) * N, D), epk3, inv16)

    return _combine(p0_2, p1_2, h9_2[R])


# speedup vs baseline: 1.8918x; 1.0005x over previous
import dataclasses
import functools

import jax
import jax.numpy as jnp
from jax import lax
from jax.experimental import pallas as pl
from jax.experimental.pallas import tpu as pltpu
from jax.experimental.pallas import tpu_sc as plsc

N = 10000
E = 320000
D = 128
R = 8
RB = 4
NC = 2
NS = 16
L = 16
NW = NC * NS

CH = 5120
CHB = 1024
RNP = NS * CH
K = 80
EPW = E // NW
NB_AGG = EPW // K
EPT_H = E // NS
NB_H = EPT_H // K
ROWS_PT = N // NS
PKB = 16384
GSPLIT = 2

BN = 1000
NBK = N // BN

f32 = jnp.float32
i32 = jnp.int32
bf16 = jnp.bfloat16

_mesh = plsc.VectorSubcoreMesh(core_axis_name="c", subcore_axis_name="s")

_sc_params = pltpu.CompilerParams()
if "needs_layout_passes" in pltpu.CompilerParams.__dataclass_fields__:
    _sc_params = dataclasses.replace(_sc_params, needs_layout_passes=False)
if "use_tc_tiling_on_sc" in pltpu.CompilerParams.__dataclass_fields__:
    _sc_params = dataclasses.replace(_sc_params, use_tc_tiling_on_sc=False)


@functools.partial(
    pl.kernel,
    out_type=jax.ShapeDtypeStruct((RNP, L), f32),
    mesh=_mesh,
    compiler_params=_sc_params,
    scratch_types=[
        pltpu.VMEM_SHARED((RNP,), f32),
        pltpu.VMEM((NB_H, K), i32),
        pltpu.VMEM((K,), f32),
        pltpu.VMEM((CH,), f32),
        pltpu.VMEM((CHB, L), f32),
        pltpu.SemaphoreType.DMA,
    ],
)
def _hist_weights(gdst3_hbm, inv16_hbm, cnt_sp, gidx2_v, ones_v, inv_v,
                  inv16_v, sem):
    cid = lax.axis_index("c")
    sid = lax.axis_index("s")

    @pl.when(cid == 0)
    def _():
        pltpu.sync_copy(gdst3_hbm.at[sid], gidx2_v)

        @pl.loop(0, CH // L)
        def _(i):
            inv_v[pl.ds(i * L, L)] = jnp.zeros((L,), f32)

        pltpu.sync_copy(inv_v, cnt_sp.at[pl.ds(sid * CH, CH)])

        @pl.loop(0, K // L)
        def _(i):
            ones_v[pl.ds(i * L, L)] = jnp.ones((L,), f32)

        plsc.subcore_barrier()

        @pl.loop(0, NB_H // 10)
        def _(i):
            for j in range(10):
                pltpu.make_async_copy(
                    ones_v, cnt_sp.at[gidx2_v.at[i * 10 + j]], sem,
                ).start(add=True)
            for j in range(10):
                pltpu.make_async_copy(
                    ones_v, cnt_sp.at[gidx2_v.at[i * 10 + j]], sem,
                ).wait()

        plsc.subcore_barrier()

        pltpu.sync_copy(cnt_sp.at[pl.ds(sid * CH, CH)], inv_v)

        @pl.loop(0, CH // L)
        def _(i):
            v = inv_v[pl.ds(i * L, L)]
            inv_v[pl.ds(i * L, L)] = 1.0 / jnp.maximum(v, 1.0)

        for c in range(CH // CHB):
            @pl.loop(0, CHB)
            def _(j):
                inv16_v[j, :] = plsc.load_gather(
                    inv_v, [jnp.full((L,), c * CHB + j, i32)])

            pltpu.sync_copy(inv16_v,
                            inv16_hbm.at[pl.ds(sid * CH + c * CHB, CHB)])


@functools.partial(
    pl.kernel,
    out_type=[jax.ShapeDtypeStruct((N, D), f32),
              jax.ShapeDtypeStruct((N, D), f32)],
    mesh=_mesh,
    compiler_params=_sc_params,
    scratch_types=[
        pltpu.VMEM_SHARED((N, D), f32),
        pltpu.VMEM((NB_AGG, K), i32),
        pltpu.VMEM((K,), i32),
        pltpu.VMEM((K,), i32),
        pltpu.VMEM((K,), i32),
        pltpu.VMEM((K,), i32),
        pltpu.VMEM((K,), i32),
        pltpu.VMEM((K,), i32),
        pltpu.VMEM((K, D), f32),
        pltpu.VMEM((K, D), f32),
        pltpu.VMEM((K, L), f32),
        pltpu.VMEM((K, L), f32),
        pltpu.SemaphoreType.DMA,
        pltpu.SemaphoreType.DMA,
        pltpu.SemaphoreType.DMA,
        pltpu.SemaphoreType.DMA,
    ],
)
def _agg(h_hbm, epk3_hbm, inv16_hbm, p0_hbm, p1_hbm,
         acc_sp, epk2_v, gidx_a, gidx_b, widx_a, widx_b, didx_a, didx_b,
         rows_a, rows_b, w16_a, w16_b, sg_a, sg_b, ss_a, ss_b):
    cid = lax.axis_index("c")
    sid = lax.axis_index("s")
    wid = cid * NS + sid

    pltpu.sync_copy(epk3_hbm.at[wid], epk2_v)

    @pl.loop(0, K)
    def _(i):
        for j in range(D // L):
            rows_a[i, pl.ds(j * L, L)] = jnp.zeros((L,), f32)

    for j in range(7):
        pltpu.sync_copy(rows_a, acc_sp.at[pl.ds(sid * ROWS_PT + j * K, K)])
    pltpu.sync_copy(rows_a, acc_sp.at[pl.ds(sid * ROWS_PT + ROWS_PT - K, K)])

    plsc.subcore_barrier()

    def mk_idx(b, gidx, widx, didx):
        @pl.loop(0, K // L)
        def _(j):
            p = epk2_v[b, pl.ds(j * L, L)]
            d = jnp.bitwise_and(p, PKB - 1)
            g = jnp.right_shift(p, 14)
            t = ((g.astype(f32) + 0.5) * (1.0 / N)).astype(i32)
            gidx[pl.ds(j * L, L)] = g
            didx[pl.ds(j * L, L)] = d
            widx[pl.ds(j * L, L)] = t * N + d

    KS = K // GSPLIT

    def g_start(gidx, widx, rows, w16, sem):
        for h in range(GSPLIT):
            pltpu.make_async_copy(h_hbm.at[gidx.at[pl.ds(h * KS, KS)]],
                                  rows.at[pl.ds(h * KS, KS)], sem).start()
        pltpu.make_async_copy(inv16_hbm.at[widx], w16, sem).start()

    def g_wait(gidx, widx, rows, w16, sem):
        for h in range(GSPLIT):
            pltpu.make_async_copy(h_hbm.at[gidx.at[pl.ds(h * KS, KS)]],
                                  rows.at[pl.ds(h * KS, KS)], sem).wait()
        pltpu.make_async_copy(inv16_hbm.at[widx], w16, sem).wait()

    def s_start(rows, didx, sem):
        pltpu.make_async_copy(rows, acc_sp.at[didx], sem).start(add=True)

    def s_wait(rows, didx, sem):
        pltpu.make_async_copy(rows, acc_sp.at[didx], sem).wait()

    def scale(rows, w16):
        @plsc.parallel_loop(0, K, unroll=4)
        def _(k):
            wv = w16[k, :]
            for j in range(D // L):
                sl = (k, pl.ds(j * L, L))
                rows[sl] = rows[sl] * wv

    mk_idx(0, gidx_a, widx_a, didx_a)
    g_start(gidx_a, widx_a, rows_a, w16_a, sg_a)
    mk_idx(1, gidx_b, widx_b, didx_b)
    g_start(gidx_b, widx_b, rows_b, w16_b, sg_b)

    @pl.loop(0, NB_AGG // 2)
    def _(i):
        b0 = 2 * i
        g_wait(gidx_a, widx_a, rows_a, w16_a, sg_a)
        scale(rows_a, w16_a)
        s_start(rows_a, didx_a, ss_a)

        g_wait(gidx_b, widx_b, rows_b, w16_b, sg_b)
        scale(rows_b, w16_b)
        s_start(rows_b, didx_b, ss_b)

        s_wait(rows_a, didx_a, ss_a)
        mk_idx(b0 + 2, gidx_a, widx_a, didx_a)
        g_start(gidx_a, widx_a, rows_a, w16_a, sg_a)

        s_wait(rows_b, didx_b, ss_b)

        @pl.when(b0 + 3 < NB_AGG)
        def _():
            mk_idx(b0 + 3, gidx_b, widx_b, didx_b)
            g_start(gidx_b, widx_b, rows_b, w16_b, sg_b)

    g_wait(gidx_a, widx_a, rows_a, w16_a, sg_a)
    scale(rows_a, w16_a)
    pltpu.sync_copy(rows_a, acc_sp.at[didx_a], add=True)

    plsc.subcore_barrier()

    @pl.when(cid == 0)
    def _():
        pltpu.sync_copy(acc_sp.at[pl.ds(sid * ROWS_PT, ROWS_PT)],
                        p0_hbm.at[pl.ds(sid * ROWS_PT, ROWS_PT)])

    @pl.when(cid == 1)
    def _():
        pltpu.sync_copy(acc_sp.at[pl.ds(sid * ROWS_PT, ROWS_PT)],
                        p1_hbm.at[pl.ds(sid * ROWS_PT, ROWS_PT)])


def _mk_w(comp_blk, basis):
    c = comp_blk[0]
    w = c[0:1, 0:1] * basis[0]
    for b in range(1, RB):
        w = w + c[0:1, b:b + 1] * basis[b]
    return w


def _prep1_body(x_ref, comp_ref, basis_ref, root_ref, bias_ref, out_ref,
                w_sc, xb_sc):
    i = pl.program_id(0)
    r = pl.program_id(1)

    @pl.when(r == 0)
    def _():
        xb_sc[...] = x_ref[...].astype(bf16)

    @pl.when((i == 0) & (r < R))
    def _():
        w_sc[r] = _mk_w(comp_ref[...], basis_ref[...]).astype(bf16)

    @pl.when(r < R)
    def _():
        out_ref[0] = jnp.dot(xb_sc[...], w_sc[r],
                             preferred_element_type=f32)

    @pl.when(r == R)
    def _():
        out_ref[0] = (jnp.dot(x_ref[...], root_ref[...],
                              preferred_element_type=f32) + bias_ref[...])


_prep1 = pl.pallas_call(
    _prep1_body,
    grid=(NBK, R + 1),
    in_specs=[
        pl.BlockSpec((BN, D), lambda i, r: (i, 0)),
        pl.BlockSpec((1, 1, RB), lambda i, r: (r, 0, 0)),
        pl.BlockSpec((RB, D, D), lambda i, r: (0, 0, 0)),
        pl.BlockSpec((D, D), lambda i, r: (0, 0)),
        pl.BlockSpec((1, D), lambda i, r: (0, 0)),
    ],
    out_specs=pl.BlockSpec((1, BN, D), lambda i, r: (r, i, 0)),
    out_shape=jax.ShapeDtypeStruct((R + 1, N, D), f32),
    scratch_shapes=[pltpu.VMEM((R, D, D), bf16),
                    pltpu.VMEM((BN, D), bf16)],
)


def _prep2_body(p0_ref, p1_ref, rt_ref, comp_ref, basis_ref, root_ref,
                bias_ref, out_ref, w_sc, h_v, hb_sc):
    i = pl.program_id(0)
    r = pl.program_id(1)

    @pl.when(r == 0)
    def _():
        h = jax.nn.sigmoid(p0_ref[...] + p1_ref[...] + rt_ref[...])
        h_v[...] = h
        hb_sc[...] = h.astype(bf16)

    @pl.when((i == 0) & (r < R))
    def _():
        w_sc[r] = _mk_w(comp_ref[...], basis_ref[...]).astype(bf16)

    @pl.when(r < R)
    def _():
        out_ref[0] = jnp.dot(hb_sc[...], w_sc[r],
                             preferred_element_type=f32)

    @pl.when(r == R)
    def _():
        out_ref[0] = (jnp.dot(h_v[...], root_ref[...],
                              preferred_element_type=f32) + bias_ref[...])


_prep2 = pl.pallas_call(
    _prep2_body,
    grid=(NBK, R + 1),
    in_specs=[
        pl.BlockSpec((BN, D), lambda i, r: (i, 0)),
        pl.BlockSpec((BN, D), lambda i, r: (i, 0)),
        pl.BlockSpec((BN, D), lambda i, r: (i, 0)),
        pl.BlockSpec((1, 1, RB), lambda i, r: (r, 0, 0)),
        pl.BlockSpec((RB, D, D), lambda i, r: (0, 0, 0)),
        pl.BlockSpec((D, D), lambda i, r: (0, 0)),
        pl.BlockSpec((1, D), lambda i, r: (0, 0)),
    ],
    out_specs=pl.BlockSpec((1, BN, D), lambda i, r: (r, i, 0)),
    out_shape=jax.ShapeDtypeStruct((R + 1, N, D), f32),
    scratch_shapes=[pltpu.VMEM((R, D, D), bf16),
                    pltpu.VMEM((BN, D), f32),
                    pltpu.VMEM((BN, D), bf16)],
)


def _combine_body(p0_ref, p1_ref, rt_ref, out_ref):
    out_ref[...] = jax.nn.sigmoid(p0_ref[...] + p1_ref[...] + rt_ref[...])


_combine = pl.pallas_call(
    _combine_body,
    grid=(NBK,),
    in_specs=[
        pl.BlockSpec((BN, D), lambda i: (i, 0)),
        pl.BlockSpec((BN, D), lambda i: (i, 0)),
        pl.BlockSpec((BN, D), lambda i: (i, 0)),
    ],
    out_specs=pl.BlockSpec((BN, D), lambda i: (i, 0)),
    out_shape=jax.ShapeDtypeStruct((N, D), f32),
)


def kernel(x, edge_index, edge_type, basis1, comp1, root1, bias1,
           basis2, comp2, root2, bias2):
    src = edge_index[0]
    dst = edge_index[1]
    gsrc = edge_type * N + src
    gdst = edge_type * N + dst
    epk3 = (gsrc * PKB + dst).reshape(NW, NB_AGG, K)
    gdst3h = gdst.reshape(NS, NB_H, K)

    inv16 = _hist_weights(gdst3h)

    pad = jnp.zeros((1, 1, RB), f32)
    comp1p = jnp.concatenate([comp1.reshape(R, 1, RB), pad], axis=0)
    comp2p = jnp.concatenate([comp2.reshape(R, 1, RB), pad], axis=0)

    h9_1 = _prep1(x, comp1p, basis1, root1, bias1.reshape(1, D))
    p0_1, p1_1 = _agg(h9_1.reshape((R + 1) * N, D), epk3, inv16)

    h9_2 = _prep2(p0_1, p1_1, h9_1[R], comp2p, basis2, root2,
                  bias2.reshape(1, D))
    p0_2, p1_2 = _agg(h9_2.reshape((R + 1) * N, D), epk3, inv16)

    return _combine(p0_2, p1_2, h9_2[R])


# speedup vs baseline: 2.0716x; 1.0951x over previous
import dataclasses
import functools

import jax
import jax.numpy as jnp
from jax import lax
from jax.experimental import pallas as pl
from jax.experimental.pallas import tpu as pltpu
from jax.experimental.pallas import tpu_sc as plsc

N = 10000
E = 320000
D = 128
R = 8
RB = 4
NC = 2
NS = 16
L = 16
NW = NC * NS

CH = 5120
CHB = 1024
RNP = NS * CH
K = 80
EPW = E // NW
NB_AGG = EPW // K
EPT_H = E // NS
NB_H = EPT_H // K
ROWS_PT = N // NS
PKB = 16384
GSPLIT = 2

BN = 2000
NBK = N // BN

f32 = jnp.float32
i32 = jnp.int32
bf16 = jnp.bfloat16

_mesh = plsc.VectorSubcoreMesh(core_axis_name="c", subcore_axis_name="s")

_sc_params = pltpu.CompilerParams()
if "needs_layout_passes" in pltpu.CompilerParams.__dataclass_fields__:
    _sc_params = dataclasses.replace(_sc_params, needs_layout_passes=False)
if "use_tc_tiling_on_sc" in pltpu.CompilerParams.__dataclass_fields__:
    _sc_params = dataclasses.replace(_sc_params, use_tc_tiling_on_sc=False)


@functools.partial(
    pl.kernel,
    out_type=jax.ShapeDtypeStruct((RNP, L), f32),
    mesh=_mesh,
    compiler_params=_sc_params,
    scratch_types=[
        pltpu.VMEM_SHARED((RNP,), f32),
        pltpu.VMEM((NB_H, K), i32),
        pltpu.VMEM((K,), f32),
        pltpu.VMEM((CH,), f32),
        pltpu.VMEM((CHB, L), f32),
        pltpu.SemaphoreType.DMA,
    ],
)
def _hist_weights(gdst3_hbm, inv16_hbm, cnt_sp, gidx2_v, ones_v, inv_v,
                  inv16_v, sem):
    cid = lax.axis_index("c")
    sid = lax.axis_index("s")

    @pl.when(cid == 0)
    def _():
        pltpu.sync_copy(gdst3_hbm.at[sid], gidx2_v)

        @pl.loop(0, CH // L)
        def _(i):
            inv_v[pl.ds(i * L, L)] = jnp.zeros((L,), f32)

        pltpu.sync_copy(inv_v, cnt_sp.at[pl.ds(sid * CH, CH)])

        @pl.loop(0, K // L)
        def _(i):
            ones_v[pl.ds(i * L, L)] = jnp.ones((L,), f32)

        plsc.subcore_barrier()

        @pl.loop(0, NB_H // 10)
        def _(i):
            for j in range(10):
                pltpu.make_async_copy(
                    ones_v, cnt_sp.at[gidx2_v.at[i * 10 + j]], sem,
                ).start(add=True)
            for j in range(10):
                pltpu.make_async_copy(
                    ones_v, cnt_sp.at[gidx2_v.at[i * 10 + j]], sem,
                ).wait()

        plsc.subcore_barrier()

        pltpu.sync_copy(cnt_sp.at[pl.ds(sid * CH, CH)], inv_v)

        @pl.loop(0, CH // L)
        def _(i):
            v = inv_v[pl.ds(i * L, L)]
            inv_v[pl.ds(i * L, L)] = 1.0 / jnp.maximum(v, 1.0)

        for c in range(CH // CHB):
            @plsc.parallel_loop(0, CHB, unroll=4)
            def _(j):
                inv16_v[j, :] = plsc.load_gather(
                    inv_v, [jnp.full((L,), c * CHB + j, i32)])

            pltpu.sync_copy(inv16_v,
                            inv16_hbm.at[pl.ds(sid * CH + c * CHB, CHB)])


@functools.partial(
    pl.kernel,
    out_type=[jax.ShapeDtypeStruct((N, D), f32),
              jax.ShapeDtypeStruct((N, D), f32)],
    mesh=_mesh,
    compiler_params=_sc_params,
    scratch_types=[
        pltpu.VMEM_SHARED((N, D), f32),
        pltpu.VMEM((NB_AGG, K), i32),
        pltpu.VMEM((K,), i32),
        pltpu.VMEM((K,), i32),
        pltpu.VMEM((K,), i32),
        pltpu.VMEM((K,), i32),
        pltpu.VMEM((K,), i32),
        pltpu.VMEM((K,), i32),
        pltpu.VMEM((K, D), f32),
        pltpu.VMEM((K, D), f32),
        pltpu.VMEM((K, L), f32),
        pltpu.VMEM((K, L), f32),
        pltpu.SemaphoreType.DMA,
        pltpu.SemaphoreType.DMA,
        pltpu.SemaphoreType.DMA,
        pltpu.SemaphoreType.DMA,
    ],
)
def _agg(h_hbm, epk3_hbm, inv16_hbm, p0_hbm, p1_hbm,
         acc_sp, epk2_v, gidx_a, gidx_b, widx_a, widx_b, didx_a, didx_b,
         rows_a, rows_b, w16_a, w16_b, sg_a, sg_b, ss_a, ss_b):
    cid = lax.axis_index("c")
    sid = lax.axis_index("s")
    wid = cid * NS + sid

    pltpu.sync_copy(epk3_hbm.at[wid], epk2_v)

    @pl.loop(0, K)
    def _(i):
        for j in range(D // L):
            rows_a[i, pl.ds(j * L, L)] = jnp.zeros((L,), f32)

    for j in range(7):
        pltpu.sync_copy(rows_a, acc_sp.at[pl.ds(sid * ROWS_PT + j * K, K)])
    pltpu.sync_copy(rows_a, acc_sp.at[pl.ds(sid * ROWS_PT + ROWS_PT - K, K)])

    plsc.subcore_barrier()

    def mk_idx(b, gidx, widx, didx):
        @pl.loop(0, K // L)
        def _(j):
            p = epk2_v[b, pl.ds(j * L, L)]
            d = jnp.bitwise_and(p, PKB - 1)
            g = jnp.right_shift(p, 14)
            t = ((g.astype(f32) + 0.5) * (1.0 / N)).astype(i32)
            gidx[pl.ds(j * L, L)] = g
            didx[pl.ds(j * L, L)] = d
            widx[pl.ds(j * L, L)] = t * N + d

    KS = K // GSPLIT

    def g_start(gidx, widx, rows, w16, sem):
        for h in range(GSPLIT):
            pltpu.make_async_copy(h_hbm.at[gidx.at[pl.ds(h * KS, KS)]],
                                  rows.at[pl.ds(h * KS, KS)], sem).start()
        pltpu.make_async_copy(inv16_hbm.at[widx], w16, sem).start()

    def g_wait(gidx, widx, rows, w16, sem):
        for h in range(GSPLIT):
            pltpu.make_async_copy(h_hbm.at[gidx.at[pl.ds(h * KS, KS)]],
                                  rows.at[pl.ds(h * KS, KS)], sem).wait()
        pltpu.make_async_copy(inv16_hbm.at[widx], w16, sem).wait()

    def s_start(rows, didx, sem):
        pltpu.make_async_copy(rows, acc_sp.at[didx], sem).start(add=True)

    def s_wait(rows, didx, sem):
        pltpu.make_async_copy(rows, acc_sp.at[didx], sem).wait()

    def scale(rows, w16):
        @plsc.parallel_loop(0, K, unroll=4)
        def _(k):
            wv = w16[k, :]
            for j in range(D // L):
                sl = (k, pl.ds(j * L, L))
                rows[sl] = rows[sl] * wv

    mk_idx(0, gidx_a, widx_a, didx_a)
    g_start(gidx_a, widx_a, rows_a, w16_a, sg_a)
    mk_idx(1, gidx_b, widx_b, didx_b)
    g_start(gidx_b, widx_b, rows_b, w16_b, sg_b)

    @pl.loop(0, NB_AGG // 2)
    def _(i):
        b0 = 2 * i
        g_wait(gidx_a, widx_a, rows_a, w16_a, sg_a)
        scale(rows_a, w16_a)
        s_start(rows_a, didx_a, ss_a)

        g_wait(gidx_b, widx_b, rows_b, w16_b, sg_b)
        scale(rows_b, w16_b)
        s_start(rows_b, didx_b, ss_b)

        s_wait(rows_a, didx_a, ss_a)
        mk_idx(b0 + 2, gidx_a, widx_a, didx_a)
        g_start(gidx_a, widx_a, rows_a, w16_a, sg_a)

        s_wait(rows_b, didx_b, ss_b)

        @pl.when(b0 + 3 < NB_AGG)
        def _():
            mk_idx(b0 + 3, gidx_b, widx_b, didx_b)
            g_start(gidx_b, widx_b, rows_b, w16_b, sg_b)

    g_wait(gidx_a, widx_a, rows_a, w16_a, sg_a)
    scale(rows_a, w16_a)
    pltpu.sync_copy(rows_a, acc_sp.at[didx_a], add=True)

    plsc.subcore_barrier()

    @pl.when(cid == 0)
    def _():
        pltpu.sync_copy(acc_sp.at[pl.ds(sid * ROWS_PT, ROWS_PT)],
                        p0_hbm.at[pl.ds(sid * ROWS_PT, ROWS_PT)])

    @pl.when(cid == 1)
    def _():
        pltpu.sync_copy(acc_sp.at[pl.ds(sid * ROWS_PT, ROWS_PT)],
                        p1_hbm.at[pl.ds(sid * ROWS_PT, ROWS_PT)])


def _mk_w(comp_blk, basis):
    c = comp_blk[0]
    w = c[0:1, 0:1] * basis[0]
    for b in range(1, RB):
        w = w + c[0:1, b:b + 1] * basis[b]
    return w


def _prep1_body(x_ref, comp_ref, basis_ref, root_ref, bias_ref, out_ref,
                w_sc, xb_sc):
    i = pl.program_id(0)
    r = pl.program_id(1)

    @pl.when(r == 0)
    def _():
        xb_sc[...] = x_ref[...].astype(bf16)

    @pl.when((i == 0) & (r < R))
    def _():
        w_sc[r] = _mk_w(comp_ref[...], basis_ref[...]).astype(bf16)

    @pl.when(r < R)
    def _():
        out_ref[0] = jnp.dot(xb_sc[...], w_sc[r],
                             preferred_element_type=f32)

    @pl.when(r == R)
    def _():
        out_ref[0] = (jnp.dot(x_ref[...], root_ref[...],
                              preferred_element_type=f32) + bias_ref[...])


_prep1 = pl.pallas_call(
    _prep1_body,
    grid=(NBK, R + 1),
    in_specs=[
        pl.BlockSpec((BN, D), lambda i, r: (i, 0)),
        pl.BlockSpec((1, 1, RB), lambda i, r: (r, 0, 0)),
        pl.BlockSpec((RB, D, D), lambda i, r: (0, 0, 0)),
        pl.BlockSpec((D, D), lambda i, r: (0, 0)),
        pl.BlockSpec((1, D), lambda i, r: (0, 0)),
    ],
    out_specs=pl.BlockSpec((1, BN, D), lambda i, r: (r, i, 0)),
    out_shape=jax.ShapeDtypeStruct((R + 1, N, D), f32),
    scratch_shapes=[pltpu.VMEM((R, D, D), bf16),
                    pltpu.VMEM((BN, D), bf16)],
)


def _prep2_body(p0_ref, p1_ref, rt_ref, comp_ref, basis_ref, root_ref,
                bias_ref, out_ref, w_sc, h_v, hb_sc):
    i = pl.program_id(0)
    r = pl.program_id(1)

    @pl.when(r == 0)
    def _():
        h = jax.nn.sigmoid(p0_ref[...] + p1_ref[...] + rt_ref[...])
        h_v[...] = h
        hb_sc[...] = h.astype(bf16)

    @pl.when((i == 0) & (r < R))
    def _():
        w_sc[r] = _mk_w(comp_ref[...], basis_ref[...]).astype(bf16)

    @pl.when(r < R)
    def _():
        out_ref[0] = jnp.dot(hb_sc[...], w_sc[r],
                             preferred_element_type=f32)

    @pl.when(r == R)
    def _():
        out_ref[0] = (jnp.dot(h_v[...], root_ref[...],
                              preferred_element_type=f32) + bias_ref[...])


_prep2 = pl.pallas_call(
    _prep2_body,
    grid=(NBK, R + 1),
    in_specs=[
        pl.BlockSpec((BN, D), lambda i, r: (i, 0)),
        pl.BlockSpec((BN, D), lambda i, r: (i, 0)),
        pl.BlockSpec((BN, D), lambda i, r: (i, 0)),
        pl.BlockSpec((1, 1, RB), lambda i, r: (r, 0, 0)),
        pl.BlockSpec((RB, D, D), lambda i, r: (0, 0, 0)),
        pl.BlockSpec((D, D), lambda i, r: (0, 0)),
        pl.BlockSpec((1, D), lambda i, r: (0, 0)),
    ],
    out_specs=pl.BlockSpec((1, BN, D), lambda i, r: (r, i, 0)),
    out_shape=jax.ShapeDtypeStruct((R + 1, N, D), f32),
    scratch_shapes=[pltpu.VMEM((R, D, D), bf16),
                    pltpu.VMEM((BN, D), f32),
                    pltpu.VMEM((BN, D), bf16)],
)


def _combine_body(p0_ref, p1_ref, rt_ref, out_ref):
    out_ref[...] = jax.nn.sigmoid(p0_ref[...] + p1_ref[...] + rt_ref[...])


_combine = pl.pallas_call(
    _combine_body,
    grid=(NBK,),
    in_specs=[
        pl.BlockSpec((BN, D), lambda i: (i, 0)),
        pl.BlockSpec((BN, D), lambda i: (i, 0)),
        pl.BlockSpec((BN, D), lambda i: (i, 0)),
    ],
    out_specs=pl.BlockSpec((BN, D), lambda i: (i, 0)),
    out_shape=jax.ShapeDtypeStruct((N, D), f32),
)


def kernel(x, edge_index, edge_type, basis1, comp1, root1, bias1,
           basis2, comp2, root2, bias2):
    src = edge_index[0]
    dst = edge_index[1]
    gsrc = edge_type * N + src
    gdst = edge_type * N + dst
    epk3 = (gsrc * PKB + dst).reshape(NW, NB_AGG, K)
    gdst3h = gdst.reshape(NS, NB_H, K)

    inv16 = _hist_weights(gdst3h)

    pad = jnp.zeros((1, 1, RB), f32)
    comp1p = jnp.concatenate([comp1.reshape(R, 1, RB), pad], axis=0)
    comp2p = jnp.concatenate([comp2.reshape(R, 1, RB), pad], axis=0)

    h9_1 = _prep1(x, comp1p, basis1, root1, bias1.reshape(1, D))
    p0_1, p1_1 = _agg(h9_1.reshape((R + 1) * N, D), epk3, inv16)

    h9_2 = _prep2(p0_1, p1_1, h9_1[R], comp2p, basis2, root2,
                  bias2.reshape(1, D))
    p0_2, p1_2 = _agg(h9_2.reshape((R + 1) * N, D), epk3, inv16)

    return _combine(p0_2, p1_2, h9_2[R])
